# baseline (speedup 1.0000x reference)
import jax
import jax.numpy as jnp
from jax import lax
from jax.experimental import pallas as pl
from jax.experimental.pallas import tpu as pltpu
from jax.experimental.pallas import tpu_sc as plsc

N = 10000
U = 256
C = 256
C4 = 64
K = 16

NW = 32
PB = 320
NP = NW * PB
BLK = PB
NB_A = NP // BLK
G1 = 16
G2 = 16
L16 = 16

f32 = jnp.float32
i32 = jnp.int32

_PIB = lax.GatherScatterMode.PROMISE_IN_BOUNDS
_DNUMS = lax.GatherDimensionNumbers(
    offset_dims=(), collapsed_slice_dims=(0,), start_index_map=(0,))

_BITREV = (0, 8, 4, 12, 2, 10, 6, 14, 1, 9, 5, 13, 3, 11, 7, 15)


def _permute(v, idx):
    return lax.gather(v, idx, _DNUMS, (1,), mode=_PIB)


def _iota16():
    return lax.iota(i32, L16)


def _zero16():
    return lax.convert_element_type(_iota16() * 0, f32)


def _lane_bcast(v, i):
    idx = ((_iota16() * 0) + i).reshape(L16, 1)
    return _permute(v, idx)


def _transpose_reduce(vs):
    iota = _iota16()
    cur = [vs[_BITREV[i]] for i in range(L16)]
    for w in (8, 4, 2, 1):
        idx = (iota ^ w).reshape(L16, 1)
        mask = (iota & w) == 0
        nxt = []
        for i in range(0, len(cur), 2):
            a, b = cur[i], cur[i + 1]
            ha = a + _permute(a, idx)
            hb = b + _permute(b, idx)
            nxt.append(jnp.where(mask, ha, hb))
        cur = nxt
    return cur[0]


def _hreduce(v, op):
    iota = _iota16()
    for w in (8, 4, 2, 1):
        v = op(v, _permute(v, (iota ^ w).reshape(L16, 1)))
    return v


def _tc_a_body(xr, yr_, wl, bl, wq, wq2, wv, bv_, xc_o, xq_o, yk_o, yv_o):
    dn = (((1,), (1,)), ((), ()))
    xc = lax.dot_general(xr[...], wl[...], dn, preferred_element_type=f32) + bl[...]
    xc_o[...] = xc
    xq_o[...] = lax.dot_general(xc, wq[...], dn, preferred_element_type=f32)
    yc = lax.dot_general(yr_[...], wl[...], dn, preferred_element_type=f32) + bl[...]
    yk_o[...] = lax.dot_general(yc, wq2[...], dn, preferred_element_type=f32)
    yv_o[...] = lax.dot_general(yc, wv[...], dn, preferred_element_type=f32) + bv_[...]


def _tc_a(xp, yp, w_lin, b_lin, wq, wq2, wv, bv):
    return pl.pallas_call(
        _tc_a_body,
        grid=(NB_A,),
        in_specs=[
            pl.BlockSpec((BLK, U), lambda j: (j, 0)),
            pl.BlockSpec((BLK, U), lambda j: (j, 0)),
            pl.BlockSpec((C, U), lambda j: (0, 0)),
            pl.BlockSpec((1, C), lambda j: (0, 0)),
            pl.BlockSpec((C4, C), lambda j: (0, 0)),
            pl.BlockSpec((128, C), lambda j: (0, 0)),
            pl.BlockSpec((C, C), lambda j: (0, 0)),
            pl.BlockSpec((1, C), lambda j: (0, 0)),
        ],
        out_specs=[
            pl.BlockSpec((BLK, C), lambda j: (j, 0)),
            pl.BlockSpec((BLK, C4), lambda j: (j, 0)),
            pl.BlockSpec((BLK, 128), lambda j: (j, 0)),
            pl.BlockSpec((BLK, C), lambda j: (j, 0)),
        ],
        out_shape=[
            jax.ShapeDtypeStruct((NP, C), f32),
            jax.ShapeDtypeStruct((NP, C4), f32),
            jax.ShapeDtypeStruct((NP, 128), f32),
            jax.ShapeDtypeStruct((NP, C), f32),
        ],
    )(xp, yp, w_lin, b_lin, wq, wq2, wv, bv)


def _sc1_body(xq_h, idx_h, yk_h, attt_h, spart_h, xqv, idxv, rows_a, rows_b,
              attv, sacc, sem_a, sem_b):
    wid = lax.axis_index("s") * 2 + lax.axis_index("c")
    base = wid * PB
    pltpu.sync_copy(xq_h.at[pl.ds(base, PB)], xqv)
    pltpu.sync_copy(idx_h.at[pl.ds(base * K, PB * K)], idxv)
    nb = PB // G1

    sacc[0, pl.ds(0, L16)] = _zero16()

    def _cps(g, rows, sem):
        i0 = g * (G1 * K)
        return (pltpu.make_async_copy(yk_h.at[idxv.at[pl.ds(i0, 128)]],
                                      rows.at[pl.ds(0, 128)], sem),
                pltpu.make_async_copy(yk_h.at[idxv.at[pl.ds(i0 + 128, 128)]],
                                      rows.at[pl.ds(128, 128)], sem))

    def issue(g, rows, sem):
        for cp in _cps(g, rows, sem):
            cp.start()

    def wait(g, rows, sem):
        for cp in _cps(g, rows, sem):
            cp.wait()

    def compute(g, rows):
        def nbody(i, c2):
            nloc = g * G1 + i
            xn = [xqv[nloc, pl.ds(cs * L16, L16)] for cs in range(C4 // L16)]
            paccs = []
            for k in range(K):
                r = i * K + k
                p = xn[0] * rows[r, pl.ds(0, L16)]
                for cs in range(1, C4 // L16):
                    p = p + xn[cs] * rows[r, pl.ds(cs * L16, L16)]
                paccs.append(p)
            e = _transpose_reduce(paccs)
            m = _hreduce(e, jnp.maximum)
            ex = jnp.exp(e - m)
            s = _hreduce(ex, lax.add)
            att = ex / s
            attv[0, pl.ds(nloc * K, L16)] = att
            vscale = jnp.where(base + nloc < N, 1.0, 0.0)
            sacc[0, pl.ds(0, L16)] = sacc[0, pl.ds(0, L16)] + att * vscale
            return c2

        lax.fori_loop(0, G1, nbody, 0)

    issue(0, rows_a, sem_a)

    def pair(p, carry):
        g0 = 2 * p
        g1 = 2 * p + 1
        wait(g0, rows_a, sem_a)
        issue(g1, rows_b, sem_b)
        compute(g0, rows_a)
        wait(g1, rows_b, sem_b)
        issue(jnp.minimum(2 * p + 2, nb - 1), rows_a, sem_a)
        compute(g1, rows_b)
        return carry

    lax.fori_loop(0, nb // 2, pair, 0)
    wait(nb - 1, rows_a, sem_a)
    pltpu.sync_copy(attv, attt_h.at[wid])
    pltpu.sync_copy(sacc, spart_h.at[wid])


def _sc1(xq, idxf, yk):
    mesh = plsc.VectorSubcoreMesh(core_axis_name="c", subcore_axis_name="s")
    return pl.kernel(
        _sc1_body,
        out_type=(
            jax.ShapeDtypeStruct((NW, 1, PB * K), f32),
            jax.ShapeDtypeStruct((NW, 1, 128), f32),
        ),
        mesh=mesh,
        scratch_types=(
            pltpu.VMEM((PB, C4), f32),
            pltpu.VMEM((PB * K,), i32),
            pltpu.VMEM((G1 * K, 128), f32),
            pltpu.VMEM((G1 * K, 128), f32),
            pltpu.VMEM((1, PB * K), f32),
            pltpu.VMEM((1, 128), f32),
            pltpu.SemaphoreType.DMA,
            pltpu.SemaphoreType.DMA,
        ),
    )(xq, idxf, yk)


def _sc2_body(attt_h, spart_h, idx_h, yv_h, yr_h, attv, idxv, spartv, rows_a,
              rows_b, yrv, sem_a, sem_b):
    wid = lax.axis_index("s") * 2 + lax.axis_index("c")
    base = wid * PB
    pltpu.sync_copy(attt_h.at[wid], attv)
    pltpu.sync_copy(idx_h.at[pl.ds(base * K, PB * K)], idxv)
    pltpu.sync_copy(spart_h, spartv)
    nb = PB // G2

    acc = spartv[0, 0, pl.ds(0, L16)]
    for w in range(1, NW):
        acc = acc + spartv[w, 0, pl.ds(0, L16)]
    rcpv = 1.0 / (1e-9 + acc)

    def _cps(g, rows, sem):
        i0 = g * (G2 * K)
        return (pltpu.make_async_copy(yv_h.at[idxv.at[pl.ds(i0, 128)]],
                                      rows.at[pl.ds(0, 128)], sem),
                pltpu.make_async_copy(yv_h.at[idxv.at[pl.ds(i0 + 128, 128)]],
                                      rows.at[pl.ds(128, 128)], sem))

    def issue(g, rows, sem):
        for cp in _cps(g, rows, sem):
            cp.start()

    def wait(g, rows, sem):
        for cp in _cps(g, rows, sem):
            cp.wait()

    def compute(g, rows):
        def nbody(i, c2):
            nloc = g * G2 + i
            wn = attv[0, pl.ds(nloc * K, L16)] * rcpv
            acc_e = [None] * (C // 32)
            acc_o = [None] * (C // 32)
            for k in range(K):
                wb = _lane_bcast(wn, k)
                r = i * K + k
                for cc in range(C // 32):
                    u = lax.bitcast_convert_type(rows[r, pl.ds(cc * L16, L16)],
                                                 i32)
                    e = lax.bitcast_convert_type(u << 16, f32)
                    o = lax.bitcast_convert_type(u & (-65536), f32)
                    te = wb * e
                    to = wb * o
                    acc_e[cc] = te if acc_e[cc] is None else acc_e[cc] + te
                    acc_o[cc] = to if acc_o[cc] is None else acc_o[cc] + to
            for cc in range(C // 32):
                eu = lax.bitcast_convert_type(acc_e[cc], i32)
                ou = lax.bitcast_convert_type(acc_o[cc], i32)
                w = (((eu + 32768) >> 16) & 65535) | ((ou + 32768) & (-65536))
                yrv[i, pl.ds(cc * L16, L16)] = lax.bitcast_convert_type(w, f32)
            return c2

        lax.fori_loop(0, G2, nbody, 0)
        pltpu.sync_copy(yrv, yr_h.at[pl.ds(base + g * G2, G2)])

    issue(0, rows_a, sem_a)

    def pair(p, carry):
        g0 = 2 * p
        g1 = 2 * p + 1
        wait(g0, rows_a, sem_a)
        issue(g1, rows_b, sem_b)
        compute(g0, rows_a)
        wait(g1, rows_b, sem_b)
        issue(jnp.minimum(2 * p + 2, nb - 1), rows_a, sem_a)
        compute(g1, rows_b)
        return carry

    lax.fori_loop(0, nb // 2, pair, 0)
    wait(nb - 1, rows_a, sem_a)


def _sc2(attt, spart, idxf, yv):
    mesh = plsc.VectorSubcoreMesh(core_axis_name="c", subcore_axis_name="s")
    return pl.kernel(
        _sc2_body,
        out_type=jax.ShapeDtypeStruct((NP, C // 2), f32),
        mesh=mesh,
        scratch_types=(
            pltpu.VMEM((1, PB * K), f32),
            pltpu.VMEM((PB * K,), i32),
            pltpu.VMEM((NW, 1, 128), f32),
            pltpu.VMEM((G2 * K, C // 2), f32),
            pltpu.VMEM((G2 * K, C // 2), f32),
            pltpu.VMEM((G2, C // 2), f32),
            pltpu.SemaphoreType.DMA,
            pltpu.SemaphoreType.DMA,
        ),
    )(attt, spart, idxf, yv)


def _tc_b_body(xc_r, yr_r, wt_r, bt_r, g_r, b_r, out_r, rbuf, ssum, ssq):
    p = pl.program_id(0)
    j = pl.program_id(1)
    dn = (((1,), (1,)), ((), ()))

    @pl.when(p == 0)
    def _phase0():
        rb = lax.dot_general(xc_r[...] - yr_r[...].astype(f32), wt_r[...], dn,
                             preferred_element_type=f32) + bt_r[...]
        rbuf[pl.ds(j * 1000, 1000), :] = rb

        @pl.when(j == 0)
        def _init():
            ssum[...] = jnp.zeros((1, C), f32)
            ssq[...] = jnp.zeros((1, C), f32)

        ssum[...] += jnp.sum(rb, axis=0, keepdims=True)
        ssq[...] += jnp.sum(rb * rb, axis=0, keepdims=True)

    @pl.when(p == 1)
    def _phase1():
        mean = ssum[...] * (1.0 / N)
        var = ssq[...] * (1.0 / N) - mean * mean
        rstd = lax.rsqrt(var + 1e-5)
        rb = rbuf[pl.ds(j * 1000, 1000), :]
        rn = (rb - mean) * rstd * g_r[...] + b_r[...]
        out_r[...] = xc_r[...] + jnp.maximum(rn, 0.0)


def _tc_b(xc, yr, wt, bt, gamma, beta):
    nb = N // 1000
    return pl.pallas_call(
        _tc_b_body,
        grid=(2, nb),
        in_specs=[
            pl.BlockSpec((1000, C), lambda p, j: (j, 0)),
            pl.BlockSpec((1000, C), lambda p, j: (j, 0)),
            pl.BlockSpec((C, C), lambda p, j: (0, 0)),
            pl.BlockSpec((1, C), lambda p, j: (0, 0)),
            pl.BlockSpec((1, C), lambda p, j: (0, 0)),
            pl.BlockSpec((1, C), lambda p, j: (0, 0)),
        ],
        out_specs=pl.BlockSpec((1000, C), lambda p, j: (j, 0)),
        out_shape=jax.ShapeDtypeStruct((N, C), f32),
        scratch_shapes=[
            pltpu.VMEM((N, C), f32),
            pltpu.VMEM((1, C), f32),
            pltpu.VMEM((1, C), f32),
        ],
    )(xc, yr, wt, bt, gamma, beta)


def kernel(x, y, neighbors_idx, W_lin, b_lin, Wq, Wv, bv, Wt, bt, gamma, beta):
    x2 = x.reshape(N, U)
    y2 = y.reshape(N, U)
    xp = jnp.pad(x2, ((0, NP - N), (0, 0)))
    yp = jnp.pad(y2, ((0, NP - N), (0, 0)))
    idxf = jnp.pad(neighbors_idx.astype(i32), ((0, NP - N), (0, 0))).reshape(-1)
    wq2 = jnp.pad(Wq, ((0, 128 - C4), (0, 0)))

    xc, xq, yk, yv = _tc_a(xp, yp, W_lin, b_lin.reshape(1, C), Wq, wq2, Wv,
                           bv.reshape(1, C))
    yv_p = lax.bitcast_convert_type(
        yv.astype(jnp.bfloat16).reshape(NP, C // 2, 2), f32)
    attt, spart = _sc1(xq, idxf, yk)
    yr_p = _sc2(attt, spart, idxf, yv_p)
    yr = lax.bitcast_convert_type(yr_p, jnp.bfloat16).reshape(NP, C)
    out2 = _tc_b(xc, yr, Wt, bt.reshape(1, C), gamma.reshape(1, C),
                 beta.reshape(1, C))
    return out2[:, :, None]


# speedup vs baseline: 2.3745x; 2.3745x over previous
import jax
import jax.numpy as jnp
from jax import lax
from jax.experimental import pallas as pl
from jax.experimental.pallas import tpu as pltpu
from jax.experimental.pallas import tpu_sc as plsc

N = 10000
U = 256
C = 256
C4 = 64
K = 16

NW = 32
PB = 320
NP = NW * PB
BLK = PB
NB_A = NP // BLK
G1 = 16
G2 = 16
L16 = 16

f32 = jnp.float32
i32 = jnp.int32

_PIB = lax.GatherScatterMode.PROMISE_IN_BOUNDS
_DNUMS = lax.GatherDimensionNumbers(
    offset_dims=(), collapsed_slice_dims=(0,), start_index_map=(0,))

_BITREV = (0, 8, 4, 12, 2, 10, 6, 14, 1, 9, 5, 13, 3, 11, 7, 15)


def _permute(v, idx):
    return lax.gather(v, idx, _DNUMS, (1,), mode=_PIB)


def _iota16():
    return lax.iota(i32, L16)


def _zero16():
    return lax.convert_element_type(_iota16() * 0, f32)


def _lane_bcast(v, i):
    idx = ((_iota16() * 0) + i).reshape(L16, 1)
    return _permute(v, idx)


def _transpose_reduce(vs):
    iota = _iota16()
    cur = [vs[_BITREV[i]] for i in range(L16)]
    for w in (8, 4, 2, 1):
        idx = (iota ^ w).reshape(L16, 1)
        mask = (iota & w) == 0
        nxt = []
        for i in range(0, len(cur), 2):
            a, b = cur[i], cur[i + 1]
            ha = a + _permute(a, idx)
            hb = b + _permute(b, idx)
            nxt.append(jnp.where(mask, ha, hb))
        cur = nxt
    return cur[0]


def _hreduce(v, op):
    iota = _iota16()
    for w in (8, 4, 2, 1):
        v = op(v, _permute(v, (iota ^ w).reshape(L16, 1)))
    return v


def _tc_a_body(xr, yr_, wl, bl, wq, wq2, wv, bv_, xc_o, xq_o, yk_o, yv_o):
    dn = (((1,), (1,)), ((), ()))
    xc = lax.dot_general(xr[...], wl[...], dn, preferred_element_type=f32) + bl[...]
    xc_o[...] = xc
    xq_o[...] = lax.dot_general(xc, wq[...], dn, preferred_element_type=f32)
    yc = lax.dot_general(yr_[...], wl[...], dn, preferred_element_type=f32) + bl[...]
    yk_o[...] = lax.dot_general(yc, wq2[...], dn, preferred_element_type=f32)
    yv_o[...] = lax.dot_general(yc, wv[...], dn, preferred_element_type=f32) + bv_[...]


def _tc_a(xp, yp, w_lin, b_lin, wq, wq2, wv, bv):
    return pl.pallas_call(
        _tc_a_body,
        grid=(NB_A,),
        in_specs=[
            pl.BlockSpec((BLK, U), lambda j: (j, 0)),
            pl.BlockSpec((BLK, U), lambda j: (j, 0)),
            pl.BlockSpec((C, U), lambda j: (0, 0)),
            pl.BlockSpec((1, C), lambda j: (0, 0)),
            pl.BlockSpec((C4, C), lambda j: (0, 0)),
            pl.BlockSpec((128, C), lambda j: (0, 0)),
            pl.BlockSpec((C, C), lambda j: (0, 0)),
            pl.BlockSpec((1, C), lambda j: (0, 0)),
        ],
        out_specs=[
            pl.BlockSpec((BLK, C), lambda j: (j, 0)),
            pl.BlockSpec((BLK, C4), lambda j: (j, 0)),
            pl.BlockSpec((BLK, 128), lambda j: (j, 0)),
            pl.BlockSpec((BLK, C), lambda j: (j, 0)),
        ],
        out_shape=[
            jax.ShapeDtypeStruct((NP, C), f32),
            jax.ShapeDtypeStruct((NP, C4), f32),
            jax.ShapeDtypeStruct((NP, 128), f32),
            jax.ShapeDtypeStruct((NP, C), f32),
        ],
    )(xp, yp, w_lin, b_lin, wq, wq2, wv, bv)


def _sc1_body(xq_h, idx_h, yk_h, attt_h, spart_h, xqv, idxv, rows_a, rows_b,
              attv, sacc, sem_a, sem_b):
    wid = lax.axis_index("s") * 2 + lax.axis_index("c")
    base = wid * PB
    pltpu.sync_copy(xq_h.at[pl.ds(base, PB)], xqv)
    pltpu.sync_copy(idx_h.at[pl.ds(base * K, PB * K)], idxv)
    nb = PB // G1

    sacc[0, pl.ds(0, L16)] = _zero16()

    def _cps(g, rows, sem):
        i0 = g * (G1 * K)
        return (pltpu.make_async_copy(yk_h.at[idxv.at[pl.ds(i0, 128)]],
                                      rows.at[pl.ds(0, 128)], sem),
                pltpu.make_async_copy(yk_h.at[idxv.at[pl.ds(i0 + 128, 128)]],
                                      rows.at[pl.ds(128, 128)], sem))

    def issue(g, rows, sem):
        for cp in _cps(g, rows, sem):
            cp.start()

    def wait(g, rows, sem):
        for cp in _cps(g, rows, sem):
            cp.wait()

    def compute(g, rows):
        def nbody(i, c2):
            nloc = g * G1 + i
            xn = [xqv[nloc, pl.ds(cs * L16, L16)] for cs in range(C4 // L16)]
            paccs = []
            for k in range(K):
                r = i * K + k
                p = xn[0] * rows[r, pl.ds(0, L16)]
                for cs in range(1, C4 // L16):
                    p = p + xn[cs] * rows[r, pl.ds(cs * L16, L16)]
                paccs.append(p)
            e = _transpose_reduce(paccs)
            m = _hreduce(e, jnp.maximum)
            ex = jnp.exp(e - m)
            s = _hreduce(ex, lax.add)
            att = ex / s
            attv[0, pl.ds(nloc * K, L16)] = att
            vscale = jnp.where(base + nloc < N, 1.0, 0.0)
            sacc[0, pl.ds(0, L16)] = sacc[0, pl.ds(0, L16)] + att * vscale
            return c2

        lax.fori_loop(0, G1, nbody, 0)

    issue(0, rows_a, sem_a)

    def pair(p, carry):
        g0 = 2 * p
        g1 = 2 * p + 1
        wait(g0, rows_a, sem_a)
        issue(g1, rows_b, sem_b)
        compute(g0, rows_a)
        wait(g1, rows_b, sem_b)
        issue(jnp.minimum(2 * p + 2, nb - 1), rows_a, sem_a)
        compute(g1, rows_b)
        return carry

    lax.fori_loop(0, nb // 2, pair, 0)
    wait(nb - 1, rows_a, sem_a)
    pltpu.sync_copy(attv, attt_h.at[wid])
    pltpu.sync_copy(sacc, spart_h.at[wid])


def _sc1(xq, idxf, yk):
    mesh = plsc.VectorSubcoreMesh(core_axis_name="c", subcore_axis_name="s")
    return pl.kernel(
        _sc1_body,
        out_type=(
            jax.ShapeDtypeStruct((NW, 1, PB * K), f32),
            jax.ShapeDtypeStruct((NW, 1, 128), f32),
        ),
        mesh=mesh,
        scratch_types=(
            pltpu.VMEM((PB, C4), f32),
            pltpu.VMEM((PB * K,), i32),
            pltpu.VMEM((G1 * K, 128), f32),
            pltpu.VMEM((G1 * K, 128), f32),
            pltpu.VMEM((1, PB * K), f32),
            pltpu.VMEM((1, 128), f32),
            pltpu.SemaphoreType.DMA,
            pltpu.SemaphoreType.DMA,
        ),
    )(xq, idxf, yk)


def _sc2_body(attt_h, spart_h, idx_h, yv_h, yr_h, attv, idxv, spartv, rows_a,
              rows_b, yrv, sem_a, sem_b):
    wid = lax.axis_index("s") * 2 + lax.axis_index("c")
    base = wid * PB
    pltpu.sync_copy(attt_h.at[wid], attv)
    pltpu.sync_copy(idx_h.at[pl.ds(base * K, PB * K)], idxv)
    pltpu.sync_copy(spart_h, spartv)
    nb = PB // G2

    acc = spartv[0, 0, pl.ds(0, L16)]
    for w in range(1, NW):
        acc = acc + spartv[w, 0, pl.ds(0, L16)]
    rcpv = 1.0 / (1e-9 + acc)

    def _cps(g, rows, sem):
        i0 = g * (G2 * K)
        return (pltpu.make_async_copy(yv_h.at[idxv.at[pl.ds(i0, 128)]],
                                      rows.at[pl.ds(0, 128)], sem),
                pltpu.make_async_copy(yv_h.at[idxv.at[pl.ds(i0 + 128, 128)]],
                                      rows.at[pl.ds(128, 128)], sem))

    def issue(g, rows, sem):
        for cp in _cps(g, rows, sem):
            cp.start()

    def wait(g, rows, sem):
        for cp in _cps(g, rows, sem):
            cp.wait()

    def compute(g, rows):
        def nbody(i, c2):
            nloc = g * G2 + i
            wn = attv[0, pl.ds(nloc * K, L16)] * rcpv
            acc_e = [None] * (C // 32)
            acc_o = [None] * (C // 32)
            for k in range(K):
                wb = _lane_bcast(wn, k)
                r = i * K + k
                for cc in range(C // 32):
                    u = lax.bitcast_convert_type(rows[r, pl.ds(cc * L16, L16)],
                                                 i32)
                    e = lax.bitcast_convert_type(u << 16, f32)
                    o = lax.bitcast_convert_type(u & (-65536), f32)
                    te = wb * e
                    to = wb * o
                    acc_e[cc] = te if acc_e[cc] is None else acc_e[cc] + te
                    acc_o[cc] = to if acc_o[cc] is None else acc_o[cc] + to
            for cc in range(C // 32):
                eu = lax.bitcast_convert_type(acc_e[cc], i32)
                ou = lax.bitcast_convert_type(acc_o[cc], i32)
                w = (((eu + 32768) >> 16) & 65535) | ((ou + 32768) & (-65536))
                yrv[i, pl.ds(cc * L16, L16)] = lax.bitcast_convert_type(w, f32)
            return c2

        lax.fori_loop(0, G2, nbody, 0)
        pltpu.sync_copy(yrv, yr_h.at[pl.ds(base + g * G2, G2)])

    issue(0, rows_a, sem_a)

    def pair(p, carry):
        g0 = 2 * p
        g1 = 2 * p + 1
        wait(g0, rows_a, sem_a)
        issue(g1, rows_b, sem_b)
        compute(g0, rows_a)
        wait(g1, rows_b, sem_b)
        issue(jnp.minimum(2 * p + 2, nb - 1), rows_a, sem_a)
        compute(g1, rows_b)
        return carry

    lax.fori_loop(0, nb // 2, pair, 0)
    wait(nb - 1, rows_a, sem_a)


def _sc2(attt, spart, idxf, yv):
    mesh = plsc.VectorSubcoreMesh(core_axis_name="c", subcore_axis_name="s")
    return pl.kernel(
        _sc2_body,
        out_type=jax.ShapeDtypeStruct((NP, C // 2), f32),
        mesh=mesh,
        scratch_types=(
            pltpu.VMEM((1, PB * K), f32),
            pltpu.VMEM((PB * K,), i32),
            pltpu.VMEM((NW, 1, 128), f32),
            pltpu.VMEM((G2 * K, C // 2), f32),
            pltpu.VMEM((G2 * K, C // 2), f32),
            pltpu.VMEM((G2, C // 2), f32),
            pltpu.SemaphoreType.DMA,
            pltpu.SemaphoreType.DMA,
        ),
    )(attt, spart, idxf, yv)


def _tc_b_body(xc_r, yr_r, wt_r, bt_r, g_r, b_r, out_r, rbuf, ssum, ssq):
    p = pl.program_id(0)
    j = pl.program_id(1)
    dn = (((1,), (1,)), ((), ()))

    @pl.when(p == 0)
    def _phase0():
        rb = lax.dot_general(xc_r[...] - yr_r[...].astype(f32), wt_r[...], dn,
                             preferred_element_type=f32) + bt_r[...]
        rbuf[pl.ds(j * 1000, 1000), :] = rb

        @pl.when(j == 0)
        def _init():
            ssum[...] = jnp.zeros((1, C), f32)
            ssq[...] = jnp.zeros((1, C), f32)

        ssum[...] += jnp.sum(rb, axis=0, keepdims=True)
        ssq[...] += jnp.sum(rb * rb, axis=0, keepdims=True)

    @pl.when(p == 1)
    def _phase1():
        mean = ssum[...] * (1.0 / N)
        var = ssq[...] * (1.0 / N) - mean * mean
        rstd = lax.rsqrt(var + 1e-5)
        rb = rbuf[pl.ds(j * 1000, 1000), :]
        rn = (rb - mean) * rstd * g_r[...] + b_r[...]
        out_r[...] = xc_r[...] + jnp.maximum(rn, 0.0)


def _tc_b(xc, yr, wt, bt, gamma, beta):
    nb = N // 1000
    return pl.pallas_call(
        _tc_b_body,
        grid=(2, nb),
        in_specs=[
            pl.BlockSpec((1000, C), lambda p, j: (j, 0)),
            pl.BlockSpec((1000, C), lambda p, j: (j, 0)),
            pl.BlockSpec((C, C), lambda p, j: (0, 0)),
            pl.BlockSpec((1, C), lambda p, j: (0, 0)),
            pl.BlockSpec((1, C), lambda p, j: (0, 0)),
            pl.BlockSpec((1, C), lambda p, j: (0, 0)),
        ],
        out_specs=pl.BlockSpec((1000, C), lambda p, j: (j, 0)),
        out_shape=jax.ShapeDtypeStruct((N, C), f32),
        scratch_shapes=[
            pltpu.VMEM((N, C), f32),
            pltpu.VMEM((1, C), f32),
            pltpu.VMEM((1, C), f32),
        ],
    )(xc, yr, wt, bt, gamma, beta)


def kernel(x, y, neighbors_idx, W_lin, b_lin, Wq, Wv, bv, Wt, bt, gamma, beta):
    x2 = x.reshape(N, U)
    y2 = y.reshape(N, U)
    xp = jnp.pad(x2, ((0, NP - N), (0, 0)))
    yp = jnp.pad(y2, ((0, NP - N), (0, 0)))
    pad_idx = jnp.arange((NP - N) * K, dtype=i32) % N
    idxf = jnp.concatenate([neighbors_idx.astype(i32).reshape(-1), pad_idx])
    wq2 = jnp.pad(Wq, ((0, 128 - C4), (0, 0)))

    xc, xq, yk, yv = _tc_a(xp, yp, W_lin, b_lin.reshape(1, C), Wq, wq2, Wv,
                           bv.reshape(1, C))
    yv_p = lax.bitcast_convert_type(
        yv.astype(jnp.bfloat16).reshape(NP, C // 2, 2), f32)
    attt, spart = _sc1(xq, idxf, yk)
    yr_p = _sc2(attt, spart, idxf, yv_p)
    yr = lax.bitcast_convert_type(yr_p, jnp.bfloat16).reshape(NP, C)
    out2 = _tc_b(xc, yr, Wt, bt.reshape(1, C), gamma.reshape(1, C),
                 beta.reshape(1, C))
    return out2[:, :, None]


# speedup vs baseline: 3.1924x; 1.3445x over previous
import jax
import jax.numpy as jnp
from jax import lax
from jax.experimental import pallas as pl
from jax.experimental.pallas import tpu as pltpu
from jax.experimental.pallas import tpu_sc as plsc

N = 10000
U = 256
C = 256
C4 = 64
K = 16

NW = 32
PB = 320
NP = NW * PB
BLK = 400
NB_A = N // BLK
G1 = 16
G2 = 16
L16 = 16

f32 = jnp.float32
i32 = jnp.int32

_PIB = lax.GatherScatterMode.PROMISE_IN_BOUNDS
_DNUMS = lax.GatherDimensionNumbers(
    offset_dims=(), collapsed_slice_dims=(0,), start_index_map=(0,))

_BITREV = (0, 8, 4, 12, 2, 10, 6, 14, 1, 9, 5, 13, 3, 11, 7, 15)


def _permute(v, idx):
    return lax.gather(v, idx, _DNUMS, (1,), mode=_PIB)


def _iota16():
    return lax.iota(i32, L16)


def _zero16():
    return lax.convert_element_type(_iota16() * 0, f32)


def _lane_bcast(v, i):
    idx = ((_iota16() * 0) + i).reshape(L16, 1)
    return _permute(v, idx)


def _transpose_reduce(vs):
    iota = _iota16()
    cur = [vs[_BITREV[i]] for i in range(L16)]
    for w in (8, 4, 2, 1):
        idx = (iota ^ w).reshape(L16, 1)
        mask = (iota & w) == 0
        nxt = []
        for i in range(0, len(cur), 2):
            a, b = cur[i], cur[i + 1]
            ha = a + _permute(a, idx)
            hb = b + _permute(b, idx)
            nxt.append(jnp.where(mask, ha, hb))
        cur = nxt
    return cur[0]


def _hreduce(v, op):
    iota = _iota16()
    for w in (8, 4, 2, 1):
        v = op(v, _permute(v, (iota ^ w).reshape(L16, 1)))
    return v


def _tc_a_body(xr, yr_, wl, bl, wq, wq2, wv, bv_, xc_o, xq_o, yk_o, yv_o):
    dn = (((1,), (1,)), ((), ()))
    xc = lax.dot_general(xr[...], wl[...], dn, preferred_element_type=f32) + bl[...]
    xc_o[...] = xc
    xq_o[...] = lax.dot_general(xc, wq[...], dn, preferred_element_type=f32)
    yc = lax.dot_general(yr_[...], wl[...], dn, preferred_element_type=f32) + bl[...]
    yk_o[...] = lax.dot_general(yc, wq2[...], dn, preferred_element_type=f32)
    yv_o[...] = lax.dot_general(yc, wv[...], dn, preferred_element_type=f32) + bv_[...]


def _tc_a(xp, yp, w_lin, b_lin, wq, wq2, wv, bv):
    return pl.pallas_call(
        _tc_a_body,
        grid=(NB_A,),
        in_specs=[
            pl.BlockSpec((BLK, U), lambda j: (j, 0)),
            pl.BlockSpec((BLK, U), lambda j: (j, 0)),
            pl.BlockSpec((C, U), lambda j: (0, 0)),
            pl.BlockSpec((1, C), lambda j: (0, 0)),
            pl.BlockSpec((C4, C), lambda j: (0, 0)),
            pl.BlockSpec((128, C), lambda j: (0, 0)),
            pl.BlockSpec((C, C), lambda j: (0, 0)),
            pl.BlockSpec((1, C), lambda j: (0, 0)),
        ],
        out_specs=[
            pl.BlockSpec((BLK, C), lambda j: (j, 0)),
            pl.BlockSpec((BLK, C4), lambda j: (j, 0)),
            pl.BlockSpec((BLK, 128), lambda j: (j, 0)),
            pl.BlockSpec((BLK, C), lambda j: (j, 0)),
        ],
        out_shape=[
            jax.ShapeDtypeStruct((NP, C), f32),
            jax.ShapeDtypeStruct((NP, C4), f32),
            jax.ShapeDtypeStruct((NP, 128), f32),
            jax.ShapeDtypeStruct((NP, C), f32),
        ],
    )(xp, yp, w_lin, b_lin, wq, wq2, wv, bv)


def _sc1_body(xq_h, idx_h, yk_h, attt_h, xqv, idxv, rows_a, rows_b,
              attv, sacc, sem_a, sem_b):
    wid = lax.axis_index("s") * 2 + lax.axis_index("c")
    base = wid * PB
    pltpu.sync_copy(xq_h.at[pl.ds(base, PB)], xqv)
    pltpu.sync_copy(idx_h.at[pl.ds(base * K, PB * K)], idxv)
    nb = PB // G1

    def zrow(rloc, c):
        for cs in range(C4 // L16):
            xqv[rloc, pl.ds(cs * L16, L16)] = _zero16()
        return c

    lax.fori_loop(jnp.minimum(PB, N - base), PB, zrow, 0)

    sacc[0, pl.ds(0, L16)] = _zero16()

    def _cps(g, rows, sem):
        i0 = g * (G1 * K)
        return (pltpu.make_async_copy(yk_h.at[idxv.at[pl.ds(i0, 128)]],
                                      rows.at[pl.ds(0, 128)], sem),
                pltpu.make_async_copy(yk_h.at[idxv.at[pl.ds(i0 + 128, 128)]],
                                      rows.at[pl.ds(128, 128)], sem))

    def issue(g, rows, sem):
        for cp in _cps(g, rows, sem):
            cp.start()

    def wait(g, rows, sem):
        for cp in _cps(g, rows, sem):
            cp.wait()

    def compute(g, rows):
        def nbody(i, c2):
            nloc = g * G1 + i
            xn = [xqv[nloc, pl.ds(cs * L16, L16)] for cs in range(C4 // L16)]
            paccs = []
            for k in range(K):
                r = i * K + k
                p = xn[0] * rows[r, pl.ds(0, L16)]
                for cs in range(1, C4 // L16):
                    p = p + xn[cs] * rows[r, pl.ds(cs * L16, L16)]
                paccs.append(p)
            e = _transpose_reduce(paccs)
            m = _hreduce(e, jnp.maximum)
            ex = jnp.exp(e - m)
            s = _hreduce(ex, lax.add)
            att = ex / s
            attv[0, pl.ds(nloc * K, L16)] = att
            vscale = jnp.where(base + nloc < N, 1.0, 0.0)
            sacc[0, pl.ds(0, L16)] = sacc[0, pl.ds(0, L16)] + att * vscale
            return c2

        lax.fori_loop(0, G1, nbody, 0)

    issue(0, rows_a, sem_a)

    def pair(p, carry):
        g0 = 2 * p
        g1 = 2 * p + 1
        wait(g0, rows_a, sem_a)
        issue(g1, rows_b, sem_b)
        compute(g0, rows_a)
        wait(g1, rows_b, sem_b)
        issue(jnp.minimum(2 * p + 2, nb - 1), rows_a, sem_a)
        compute(g1, rows_b)
        return carry

    lax.fori_loop(0, nb // 2, pair, 0)
    wait(nb - 1, rows_a, sem_a)
    pltpu.sync_copy(attv, attt_h.at[wid, :, pl.ds(0, PB * K)])
    pltpu.sync_copy(sacc, attt_h.at[wid, :, pl.ds(PB * K, 128)])


def _sc1(xq, idxf, yk):
    mesh = plsc.VectorSubcoreMesh(core_axis_name="c", subcore_axis_name="s")
    return pl.kernel(
        _sc1_body,
        out_type=jax.ShapeDtypeStruct((NW, 1, PB * K + 128), f32),
        mesh=mesh,
        scratch_types=(
            pltpu.VMEM((PB, C4), f32),
            pltpu.VMEM((PB * K,), i32),
            pltpu.VMEM((G1 * K, 128), f32),
            pltpu.VMEM((G1 * K, 128), f32),
            pltpu.VMEM((1, PB * K), f32),
            pltpu.VMEM((1, 128), f32),
            pltpu.SemaphoreType.DMA,
            pltpu.SemaphoreType.DMA,
        ),
    )(xq, idxf, yk)


def _sc2_body(attt_h, idx_h, yv_h, yr_h, attv, idxv, spartv, rows_a,
              rows_b, yrv, sem_a, sem_b):
    wid = lax.axis_index("s") * 2 + lax.axis_index("c")
    base = wid * PB
    pltpu.sync_copy(attt_h.at[wid, :, pl.ds(0, PB * K)], attv)
    pltpu.sync_copy(idx_h.at[pl.ds(base * K, PB * K)], idxv)
    pltpu.sync_copy(attt_h.at[:, :, pl.ds(PB * K, 128)], spartv)
    nb = PB // G2

    acc = spartv[0, 0, pl.ds(0, L16)]
    for w in range(1, NW):
        acc = acc + spartv[w, 0, pl.ds(0, L16)]
    rcpv = 1.0 / (1e-9 + acc)

    def _cps(g, rows, sem):
        i0 = g * (G2 * K)
        return (pltpu.make_async_copy(yv_h.at[idxv.at[pl.ds(i0, 128)]],
                                      rows.at[pl.ds(0, 128)], sem),
                pltpu.make_async_copy(yv_h.at[idxv.at[pl.ds(i0 + 128, 128)]],
                                      rows.at[pl.ds(128, 128)], sem))

    def issue(g, rows, sem):
        for cp in _cps(g, rows, sem):
            cp.start()

    def wait(g, rows, sem):
        for cp in _cps(g, rows, sem):
            cp.wait()

    def compute(g, rows):
        def nbody(i, c2):
            nloc = g * G2 + i
            wn = attv[0, pl.ds(nloc * K, L16)] * rcpv
            acc_e = [None] * (C // 32)
            acc_o = [None] * (C // 32)
            for k in range(K):
                wb = _lane_bcast(wn, k)
                r = i * K + k
                for cc in range(C // 32):
                    u = lax.bitcast_convert_type(rows[r, pl.ds(cc * L16, L16)],
                                                 i32)
                    e = lax.bitcast_convert_type(u << 16, f32)
                    o = lax.bitcast_convert_type(u & (-65536), f32)
                    te = wb * e
                    to = wb * o
                    acc_e[cc] = te if acc_e[cc] is None else acc_e[cc] + te
                    acc_o[cc] = to if acc_o[cc] is None else acc_o[cc] + to
            for cc in range(C // 32):
                yrv[i, pl.ds(cc * L16, L16)] = acc_e[cc]
                yrv[i, pl.ds(128 + cc * L16, L16)] = acc_o[cc]
            return c2

        lax.fori_loop(0, G2, nbody, 0)
        pltpu.sync_copy(yrv, yr_h.at[pl.ds(base + g * G2, G2)])

    issue(0, rows_a, sem_a)

    def pair(p, carry):
        g0 = 2 * p
        g1 = 2 * p + 1
        wait(g0, rows_a, sem_a)
        issue(g1, rows_b, sem_b)
        compute(g0, rows_a)
        wait(g1, rows_b, sem_b)
        issue(jnp.minimum(2 * p + 2, nb - 1), rows_a, sem_a)
        compute(g1, rows_b)
        return carry

    lax.fori_loop(0, nb // 2, pair, 0)
    wait(nb - 1, rows_a, sem_a)


def _sc2(attt, idxf, yv):
    mesh = plsc.VectorSubcoreMesh(core_axis_name="c", subcore_axis_name="s")
    return pl.kernel(
        _sc2_body,
        out_type=jax.ShapeDtypeStruct((NP, C), f32),
        mesh=mesh,
        scratch_types=(
            pltpu.VMEM((1, PB * K), f32),
            pltpu.VMEM((PB * K,), i32),
            pltpu.VMEM((NW, 1, 128), f32),
            pltpu.VMEM((G2 * K, C // 2), f32),
            pltpu.VMEM((G2 * K, C // 2), f32),
            pltpu.VMEM((G2, C), f32),
            pltpu.SemaphoreType.DMA,
            pltpu.SemaphoreType.DMA,
        ),
    )(attt, idxf, yv)


def _tc_b_body(xc_r, yr_r, wt_r, wteo_r, bt_r, g_r, b_r, out_r, rbuf, ssum, ssq):
    p = pl.program_id(0)
    j = pl.program_id(1)
    dn = (((1,), (1,)), ((), ()))

    @pl.when(p == 0)
    def _phase0():
        rb = (lax.dot_general(xc_r[...], wt_r[...], dn,
                              preferred_element_type=f32)
              - lax.dot_general(yr_r[...], wteo_r[...], dn,
                                preferred_element_type=f32)) + bt_r[...]
        rbuf[pl.ds(j * 1000, 1000), :] = rb

        @pl.when(j == 0)
        def _init():
            ssum[...] = jnp.zeros((1, C), f32)
            ssq[...] = jnp.zeros((1, C), f32)

        ssum[...] += jnp.sum(rb, axis=0, keepdims=True)
        ssq[...] += jnp.sum(rb * rb, axis=0, keepdims=True)

    @pl.when(p == 1)
    def _phase1():
        mean = ssum[...] * (1.0 / N)
        var = ssq[...] * (1.0 / N) - mean * mean
        rstd = lax.rsqrt(var + 1e-5)
        rb = rbuf[pl.ds(j * 1000, 1000), :]
        rn = (rb - mean) * rstd * g_r[...] + b_r[...]
        out_r[...] = xc_r[...] + jnp.maximum(rn, 0.0)


def _tc_b(xc, yr, wt, wteo, bt, gamma, beta):
    nb = N // 1000
    return pl.pallas_call(
        _tc_b_body,
        grid=(2, nb),
        in_specs=[
            pl.BlockSpec((1000, C), lambda p, j: (j, 0)),
            pl.BlockSpec((1000, C), lambda p, j: (j, 0)),
            pl.BlockSpec((C, C), lambda p, j: (0, 0)),
            pl.BlockSpec((C, C), lambda p, j: (0, 0)),
            pl.BlockSpec((1, C), lambda p, j: (0, 0)),
            pl.BlockSpec((1, C), lambda p, j: (0, 0)),
            pl.BlockSpec((1, C), lambda p, j: (0, 0)),
        ],
        out_specs=pl.BlockSpec((1000, C), lambda p, j: (j, 0)),
        out_shape=jax.ShapeDtypeStruct((N, C), f32),
        scratch_shapes=[
            pltpu.VMEM((N, C), f32),
            pltpu.VMEM((1, C), f32),
            pltpu.VMEM((1, C), f32),
        ],
    )(xc, yr, wt, wteo, bt, gamma, beta)


def kernel(x, y, neighbors_idx, W_lin, b_lin, Wq, Wv, bv, Wt, bt, gamma, beta):
    x2 = x.reshape(N, U)
    y2 = y.reshape(N, U)
    pad_idx = jnp.arange((NP - N) * K, dtype=i32) % N
    idxf = jnp.concatenate([neighbors_idx.astype(i32).reshape(-1), pad_idx])
    wq2 = jnp.pad(Wq, ((0, 128 - C4), (0, 0)))
    wteo = jnp.concatenate([Wt[:, 0::2], Wt[:, 1::2]], axis=1)

    xc, xq, yk, yv = _tc_a(x2, y2, W_lin, b_lin.reshape(1, C), Wq, wq2, Wv,
                           bv.reshape(1, C))
    yv_p = lax.bitcast_convert_type(
        yv.astype(jnp.bfloat16).reshape(NP, C // 2, 2), f32)
    attt = _sc1(xq, idxf, yk)
    yr = _sc2(attt, idxf, yv_p)
    out2 = _tc_b(xc, yr, Wt, wteo, bt.reshape(1, C), gamma.reshape(1, C),
                 beta.reshape(1, C))
    return out2[:, :, None]


# speedup vs baseline: 3.3980x; 1.0644x over previous
import jax
import jax.numpy as jnp
from jax import lax
from jax.experimental import pallas as pl
from jax.experimental.pallas import tpu as pltpu
from jax.experimental.pallas import tpu_sc as plsc

N = 10000
U = 256
C = 256
C4 = 64
K = 16

NW = 32
PB = 320
NP = NW * PB
BLK = 400
NB_A = N // BLK
G1 = 16
G2 = 16
L16 = 16

f32 = jnp.float32
i32 = jnp.int32

_PIB = lax.GatherScatterMode.PROMISE_IN_BOUNDS
_DNUMS = lax.GatherDimensionNumbers(
    offset_dims=(), collapsed_slice_dims=(0,), start_index_map=(0,))

_BITREV = (0, 8, 4, 12, 2, 10, 6, 14, 1, 9, 5, 13, 3, 11, 7, 15)


def _permute(v, idx):
    return lax.gather(v, idx, _DNUMS, (1,), mode=_PIB)


def _iota16():
    return lax.iota(i32, L16)


def _zero16():
    return lax.convert_element_type(_iota16() * 0, f32)


def _lane_bcast(v, i):
    idx = ((_iota16() * 0) + i).reshape(L16, 1)
    return _permute(v, idx)


def _transpose_reduce(vs):
    iota = _iota16()
    cur = [vs[_BITREV[i]] for i in range(L16)]
    for w in (8, 4, 2, 1):
        idx = (iota ^ w).reshape(L16, 1)
        mask = (iota & w) == 0
        nxt = []
        for i in range(0, len(cur), 2):
            a, b = cur[i], cur[i + 1]
            ha = a + _permute(a, idx)
            hb = b + _permute(b, idx)
            nxt.append(jnp.where(mask, ha, hb))
        cur = nxt
    return cur[0]


def _hreduce(v, op):
    iota = _iota16()
    for w in (8, 4, 2, 1):
        v = op(v, _permute(v, (iota ^ w).reshape(L16, 1)))
    return v


def _tc_a_body(xr, yr_, wl, bl, wq, wq2, wv, bv_, xc_o, xq_o, yk_o, yv_o):
    dn = (((1,), (1,)), ((), ()))
    xb = xr[...].reshape(BLK, U)
    yb = yr_[...].reshape(BLK, U)
    xc = lax.dot_general(xb, wl[...], dn, preferred_element_type=f32) + bl[...]
    xc_o[...] = xc
    xq_o[...] = lax.dot_general(xc, wq[...], dn, preferred_element_type=f32)
    yc = lax.dot_general(yb, wl[...], dn, preferred_element_type=f32) + bl[...]
    yk_o[...] = lax.dot_general(yc, wq2[...], dn, preferred_element_type=f32)
    yv_o[...] = lax.dot_general(yc, wv[...], dn, preferred_element_type=f32) + bv_[...]


def _tc_a(xp, yp, w_lin, b_lin, wq, wq2, wv, bv):
    return pl.pallas_call(
        _tc_a_body,
        grid=(NB_A,),
        in_specs=[
            pl.BlockSpec((BLK, 1, U), lambda j: (j, 0, 0)),
            pl.BlockSpec((BLK, 1, U), lambda j: (j, 0, 0)),
            pl.BlockSpec((C, U), lambda j: (0, 0)),
            pl.BlockSpec((1, C), lambda j: (0, 0)),
            pl.BlockSpec((C4, C), lambda j: (0, 0)),
            pl.BlockSpec((128, C), lambda j: (0, 0)),
            pl.BlockSpec((C, C), lambda j: (0, 0)),
            pl.BlockSpec((1, C), lambda j: (0, 0)),
        ],
        out_specs=[
            pl.BlockSpec((BLK, C), lambda j: (j, 0)),
            pl.BlockSpec((BLK, C4), lambda j: (j, 0)),
            pl.BlockSpec((BLK, 128), lambda j: (j, 0)),
            pl.BlockSpec((BLK, C), lambda j: (j, 0)),
        ],
        out_shape=[
            jax.ShapeDtypeStruct((NP, C), f32),
            jax.ShapeDtypeStruct((NP, C4), f32),
            jax.ShapeDtypeStruct((NP, 128), f32),
            jax.ShapeDtypeStruct((NP, C), f32),
        ],
    )(xp, yp, w_lin, b_lin, wq, wq2, wv, bv)


def _sc1_body(xq_h, idx_h, yk_h, attt_h, xqv, idxv, rows_a, rows_b,
              attv, sacc, sem_a, sem_b):
    wid = lax.axis_index("s") * 2 + lax.axis_index("c")
    base = wid * PB
    pltpu.sync_copy(xq_h.at[pl.ds(base, PB)], xqv)
    pltpu.sync_copy(idx_h.at[pl.ds(base * K, PB * K)], idxv)
    nb = PB // G1

    def zrow(rloc, c):
        for cs in range(C4 // L16):
            xqv[rloc, pl.ds(cs * L16, L16)] = _zero16()
        return c

    lax.fori_loop(jnp.minimum(PB, N - base), PB, zrow, 0)

    sacc[0, pl.ds(0, L16)] = _zero16()

    def _cps(g, rows, sem):
        i0 = g * (G1 * K)
        return (pltpu.make_async_copy(yk_h.at[idxv.at[pl.ds(i0, 128)]],
                                      rows.at[pl.ds(0, 128)], sem),
                pltpu.make_async_copy(yk_h.at[idxv.at[pl.ds(i0 + 128, 128)]],
                                      rows.at[pl.ds(128, 128)], sem))

    def issue(g, rows, sem):
        for cp in _cps(g, rows, sem):
            cp.start()

    def wait(g, rows, sem):
        for cp in _cps(g, rows, sem):
            cp.wait()

    def compute(g, rows):
        def nbody(i, c2):
            nloc = g * G1 + i
            xn = [xqv[nloc, pl.ds(cs * L16, L16)] for cs in range(C4 // L16)]
            paccs = []
            for k in range(K):
                r = i * K + k
                p = xn[0] * rows[r, pl.ds(0, L16)]
                for cs in range(1, C4 // L16):
                    p = p + xn[cs] * rows[r, pl.ds(cs * L16, L16)]
                paccs.append(p)
            e = _transpose_reduce(paccs)
            m = _hreduce(e, jnp.maximum)
            ex = jnp.exp(e - m)
            s = _hreduce(ex, lax.add)
            att = ex / s
            attv[0, pl.ds(nloc * K, L16)] = att
            vscale = jnp.where(base + nloc < N, 1.0, 0.0)
            sacc[0, pl.ds(0, L16)] = sacc[0, pl.ds(0, L16)] + att * vscale
            return c2

        lax.fori_loop(0, G1, nbody, 0)

    issue(0, rows_a, sem_a)

    def pair(p, carry):
        g0 = 2 * p
        g1 = 2 * p + 1
        wait(g0, rows_a, sem_a)
        issue(g1, rows_b, sem_b)
        compute(g0, rows_a)
        wait(g1, rows_b, sem_b)
        issue(jnp.minimum(2 * p + 2, nb - 1), rows_a, sem_a)
        compute(g1, rows_b)
        return carry

    lax.fori_loop(0, nb // 2, pair, 0)
    wait(nb - 1, rows_a, sem_a)
    pltpu.sync_copy(attv, attt_h.at[wid, :, pl.ds(0, PB * K)])
    pltpu.sync_copy(sacc, attt_h.at[wid, :, pl.ds(PB * K, 128)])


def _sc1(xq, idxf, yk):
    mesh = plsc.VectorSubcoreMesh(core_axis_name="c", subcore_axis_name="s")
    return pl.kernel(
        _sc1_body,
        out_type=jax.ShapeDtypeStruct((NW, 1, PB * K + 128), f32),
        mesh=mesh,
        scratch_types=(
            pltpu.VMEM((PB, C4), f32),
            pltpu.VMEM((PB * K,), i32),
            pltpu.VMEM((G1 * K, 128), f32),
            pltpu.VMEM((G1 * K, 128), f32),
            pltpu.VMEM((1, PB * K), f32),
            pltpu.VMEM((1, 128), f32),
            pltpu.SemaphoreType.DMA,
            pltpu.SemaphoreType.DMA,
        ),
    )(xq, idxf, yk)


def _sc2_body(attt_h, idx_h, yv_h, yr_h, attv, idxv, spartv, rows_a,
              rows_b, yrv, sem_a, sem_b):
    wid = lax.axis_index("s") * 2 + lax.axis_index("c")
    base = wid * PB
    pltpu.sync_copy(attt_h.at[wid, :, pl.ds(0, PB * K)], attv)
    pltpu.sync_copy(idx_h.at[pl.ds(base * K, PB * K)], idxv)
    pltpu.sync_copy(attt_h.at[:, :, pl.ds(PB * K, 128)], spartv)
    nb = PB // G2

    acc = spartv[0, 0, pl.ds(0, L16)]
    for w in range(1, NW):
        acc = acc + spartv[w, 0, pl.ds(0, L16)]
    rcpv = 1.0 / (1e-9 + acc)

    def _cps(g, rows, sem):
        i0 = g * (G2 * K)
        return (pltpu.make_async_copy(yv_h.at[idxv.at[pl.ds(i0, 128)]],
                                      rows.at[pl.ds(0, 128)], sem),
                pltpu.make_async_copy(yv_h.at[idxv.at[pl.ds(i0 + 128, 128)]],
                                      rows.at[pl.ds(128, 128)], sem))

    def issue(g, rows, sem):
        for cp in _cps(g, rows, sem):
            cp.start()

    def wait(g, rows, sem):
        for cp in _cps(g, rows, sem):
            cp.wait()

    def compute(g, rows):
        def nbody(i, c2):
            nloc = g * G2 + i
            wn = attv[0, pl.ds(nloc * K, L16)] * rcpv
            acc_e = [None] * (C // 32)
            acc_o = [None] * (C // 32)
            for k in range(K):
                wb = _lane_bcast(wn, k)
                r = i * K + k
                for cc in range(C // 32):
                    u = lax.bitcast_convert_type(rows[r, pl.ds(cc * L16, L16)],
                                                 i32)
                    e = lax.bitcast_convert_type(u << 16, f32)
                    o = lax.bitcast_convert_type(u & (-65536), f32)
                    te = wb * e
                    to = wb * o
                    acc_e[cc] = te if acc_e[cc] is None else acc_e[cc] + te
                    acc_o[cc] = to if acc_o[cc] is None else acc_o[cc] + to
            for cc in range(C // 32):
                yrv[i, pl.ds(cc * L16, L16)] = acc_e[cc]
                yrv[i, pl.ds(128 + cc * L16, L16)] = acc_o[cc]
            return c2

        lax.fori_loop(0, G2, nbody, 0)
        pltpu.sync_copy(yrv, yr_h.at[pl.ds(base + g * G2, G2)])

    issue(0, rows_a, sem_a)

    def pair(p, carry):
        g0 = 2 * p
        g1 = 2 * p + 1
        wait(g0, rows_a, sem_a)
        issue(g1, rows_b, sem_b)
        compute(g0, rows_a)
        wait(g1, rows_b, sem_b)
        issue(jnp.minimum(2 * p + 2, nb - 1), rows_a, sem_a)
        compute(g1, rows_b)
        return carry

    lax.fori_loop(0, nb // 2, pair, 0)
    wait(nb - 1, rows_a, sem_a)


def _sc2(attt, idxf, yv):
    mesh = plsc.VectorSubcoreMesh(core_axis_name="c", subcore_axis_name="s")
    return pl.kernel(
        _sc2_body,
        out_type=jax.ShapeDtypeStruct((NP, C), f32),
        mesh=mesh,
        scratch_types=(
            pltpu.VMEM((1, PB * K), f32),
            pltpu.VMEM((PB * K,), i32),
            pltpu.VMEM((NW, 1, 128), f32),
            pltpu.VMEM((G2 * K, C // 2), f32),
            pltpu.VMEM((G2 * K, C // 2), f32),
            pltpu.VMEM((G2, C), f32),
            pltpu.SemaphoreType.DMA,
            pltpu.SemaphoreType.DMA,
        ),
    )(attt, idxf, yv)


def _tc_b_body(xc_r, yr_r, wt_r, wteo_r, bt_r, g_r, b_r, out_r, rbuf, ssum, ssq):
    p = pl.program_id(0)
    j = pl.program_id(1)
    dn = (((1,), (1,)), ((), ()))

    @pl.when(p == 0)
    def _phase0():
        rb = (lax.dot_general(xc_r[...], wt_r[...], dn,
                              preferred_element_type=f32)
              - lax.dot_general(yr_r[...], wteo_r[...], dn,
                                preferred_element_type=f32)) + bt_r[...]
        rbuf[pl.ds(j * 1000, 1000), :] = rb

        @pl.when(j == 0)
        def _init():
            ssum[...] = jnp.zeros((1, C), f32)
            ssq[...] = jnp.zeros((1, C), f32)

        ssum[...] += jnp.sum(rb, axis=0, keepdims=True)
        ssq[...] += jnp.sum(rb * rb, axis=0, keepdims=True)

    @pl.when(p == 1)
    def _phase1():
        mean = ssum[...] * (1.0 / N)
        var = ssq[...] * (1.0 / N) - mean * mean
        rstd = lax.rsqrt(var + 1e-5)
        rb = rbuf[pl.ds(j * 1000, 1000), :]
        rn = (rb - mean) * rstd * g_r[...] + b_r[...]
        out_r[...] = xc_r[...] + jnp.maximum(rn, 0.0)


def _tc_b(xc, yr, wt, wteo, bt, gamma, beta):
    nb = N // 1000
    return pl.pallas_call(
        _tc_b_body,
        grid=(2, nb),
        in_specs=[
            pl.BlockSpec((1000, C), lambda p, j: (j, 0)),
            pl.BlockSpec((1000, C), lambda p, j: (j, 0)),
            pl.BlockSpec((C, C), lambda p, j: (0, 0)),
            pl.BlockSpec((C, C), lambda p, j: (0, 0)),
            pl.BlockSpec((1, C), lambda p, j: (0, 0)),
            pl.BlockSpec((1, C), lambda p, j: (0, 0)),
            pl.BlockSpec((1, C), lambda p, j: (0, 0)),
        ],
        out_specs=pl.BlockSpec((1000, C), lambda p, j: (j, 0)),
        out_shape=jax.ShapeDtypeStruct((N, C), f32),
        scratch_shapes=[
            pltpu.VMEM((N, C), f32),
            pltpu.VMEM((1, C), f32),
            pltpu.VMEM((1, C), f32),
        ],
    )(xc, yr, wt, wteo, bt, gamma, beta)


def kernel(x, y, neighbors_idx, W_lin, b_lin, Wq, Wv, bv, Wt, bt, gamma, beta):
    pad_idx = jnp.arange((NP - N) * K, dtype=i32) % N
    idxf = jnp.concatenate([neighbors_idx.astype(i32).reshape(-1), pad_idx])
    wq2 = jnp.pad(Wq, ((0, 128 - C4), (0, 0)))
    wteo = jnp.concatenate([Wt[:, 0::2], Wt[:, 1::2]], axis=1)

    xc, xq, yk, yv = _tc_a(x, y, W_lin, b_lin.reshape(1, C), Wq, wq2, Wv,
                           bv.reshape(1, C))
    yv_p = lax.bitcast_convert_type(
        yv.astype(jnp.bfloat16).reshape(NP, C // 2, 2), f32)
    attt = _sc1(xq, idxf, yk)
    yr = _sc2(attt, idxf, yv_p)
    out2 = _tc_b(xc, yr, Wt, wteo, bt.reshape(1, C), gamma.reshape(1, C),
                 beta.reshape(1, C))
    return out2[:, :, None]
